# baseline (speedup 1.0000x reference)
import functools

import jax
import jax.numpy as jnp
from jax import lax
from jax.experimental import pallas as pl
from jax.experimental.pallas import tpu as pltpu
from jax.experimental.pallas import tpu_sc as plsc

D_IN = 3
GRID = 64
D_OUT = 32
CH = 128
L = 16

_CORNER_OFFS = tuple(
    e0 * GRID * GRID + e1 * GRID + e2
    for e0 in (0, 1) for e1 in (0, 1) for e2 in (0, 1)
)


def _make_sc_interp(n_query: int):
    info = plsc.get_sparse_core_info()
    nc, ns = info.num_cores, info.num_subcores
    nw = nc * ns
    assert n_query % (nw * 2 * CH) == 0
    qpw = n_query // nw
    n_chunks = qpw // CH

    mesh = plsc.VectorSubcoreMesh(core_axis_name="c", subcore_axis_name="s")

    @functools.partial(
        pl.kernel,
        out_type=jax.ShapeDtypeStruct((n_query, D_OUT), jnp.float32),
        mesh=mesh,
        compiler_params=pltpu.CompilerParams(use_tc_tiling_on_sc=False,
                                             needs_layout_passes=False),
        scratch_types=[
            pltpu.VMEM((2, CH * D_IN), jnp.float32),
            pltpu.VMEM((2, 8, CH), jnp.int32),
            pltpu.VMEM((2, 8, CH), jnp.float32),
            pltpu.VMEM((2, 8, CH, D_OUT), jnp.float32),
            pltpu.VMEM((2, CH, D_OUT), jnp.float32),
            pltpu.SemaphoreType.DMA,
            pltpu.SemaphoreType.DMA,
            pltpu.SemaphoreType.DMA,
            pltpu.SemaphoreType.DMA,
        ],
    )
    def interp(yt, x, out, xv, idxv, wv, rowsv, outv, gs0, gs1, os0, os1):
        wid = lax.axis_index("s") * nc + lax.axis_index("c")
        wbase = wid * qpw
        gsem = (gs0, gs1)
        osem = (os0, os1)
        lane = jnp.arange(L, dtype=jnp.int32)

        def stage(ci, b):
            qbase = wbase + ci * CH
            pltpu.sync_copy(x.at[pl.ds(qbase * D_IN, CH * D_IN)], xv.at[b])

            def group_body(g, _):
                sl = pl.ds(g * L, L)
                rows = (g * L + lane) * D_IN
                f = []
                for d in range(D_IN):
                    c = plsc.load_gather(xv.at[b], [rows + d])
                    f.append(jnp.clip(c, 0.0, float(GRID - 1)))
                i0 = jnp.minimum(f[0].astype(jnp.int32), GRID - 2)
                i1 = jnp.minimum(f[1].astype(jnp.int32), GRID - 2)
                i2 = jnp.minimum(f[2].astype(jnp.int32), GRID - 2)
                t0 = f[0] - i0.astype(jnp.float32)
                t1 = f[1] - i1.astype(jnp.float32)
                t2 = f[2] - i2.astype(jnp.float32)
                u0 = 1.0 - t0
                u1 = 1.0 - t1
                u2 = 1.0 - t2
                base = i0 * (GRID * GRID) + i1 * GRID + i2
                a00 = u1 * u2
                a01 = u1 * t2
                a10 = t1 * u2
                a11 = t1 * t2
                ws = (u0 * a00, u0 * a01, u0 * a10, u0 * a11,
                      t0 * a00, t0 * a01, t0 * a10, t0 * a11)
                for c in range(8):
                    idxv[b, c, sl] = base + _CORNER_OFFS[c]
                    wv[b, c, sl] = ws[c]
                return 0

            lax.fori_loop(0, CH // L, group_body, 0)
            for c in range(8):
                pltpu.async_copy(yt.at[idxv.at[b, c]], rowsv.at[b, c], gsem[b])

        def drain_gathers(b):
            for c in range(8):
                pltpu.make_async_copy(yt.at[idxv.at[b, c]], rowsv.at[b, c],
                                      gsem[b]).wait()

        def combine(ci, b, k):
            qbase = wbase + ci * CH

            def comb_body(g, _):
                sl = pl.ds(g * L, L)
                wvecs = [wv[b, c, sl] for c in range(8)]
                for j in range(L):
                    q = g * L + j
                    w = [wvecs[c][j] for c in range(8)]
                    for h in range(D_OUT // L):
                        hs = pl.ds(h * L, L)
                        acc = w[0] * rowsv[b, 0, q, hs]
                        for c in range(1, 8):
                            acc = acc + w[c] * rowsv[b, c, q, hs]
                        outv[b, q, hs] = acc
                return 0

            drain_gathers(b)

            @pl.when(k > 0)
            def _():
                pltpu.make_async_copy(outv.at[b], out.at[pl.ds(qbase, CH)],
                                      osem[b]).wait()

            lax.fori_loop(0, CH // L, comb_body, 0)
            pltpu.async_copy(outv.at[b], out.at[pl.ds(qbase, CH)], osem[b])

        stage(0, 0)

        def pair_body(k, _):
            stage(2 * k + 1, 1)
            combine(2 * k, 0, k)

            @pl.when(k < n_chunks // 2 - 1)
            def _():
                stage(2 * k + 2, 0)

            combine(2 * k + 1, 1, k)
            return 0

        lax.fori_loop(0, n_chunks // 2, pair_body, 0)
        pltpu.make_async_copy(outv.at[0], out.at[pl.ds(wbase, CH)], os0).wait()
        pltpu.make_async_copy(outv.at[1], out.at[pl.ds(wbase, CH)], os1).wait()

    return interp


def kernel(y, xs0, xs1, xs2, x):
    n_query = x.shape[0]
    yt = y.reshape(GRID * GRID * GRID, D_OUT)
    interp = _make_sc_interp(n_query)
    return interp(yt, x.reshape(-1))


# speedup vs baseline: 1.7587x; 1.7587x over previous
import functools

import jax
import jax.numpy as jnp
from jax import lax
from jax.experimental import pallas as pl
from jax.experimental.pallas import tpu as pltpu
from jax.experimental.pallas import tpu_sc as plsc

D_IN = 3
GRID = 64
D_OUT = 32
CH = 128
L = 16

_CORNER_OFFS = tuple(
    e0 * GRID * GRID + e1 * GRID + e2
    for e0 in (0, 1) for e1 in (0, 1) for e2 in (0, 1)
)


def _make_sc_interp(n_query: int):
    info = plsc.get_sparse_core_info()
    nc, ns = info.num_cores, info.num_subcores
    nw = nc * ns
    assert n_query % (nw * 2 * CH) == 0
    qpw = n_query // nw
    n_chunks = qpw // CH

    mesh = plsc.VectorSubcoreMesh(core_axis_name="c", subcore_axis_name="s")

    @functools.partial(
        pl.kernel,
        out_type=jax.ShapeDtypeStruct((n_query, D_OUT), jnp.float32),
        mesh=mesh,
        compiler_params=pltpu.CompilerParams(use_tc_tiling_on_sc=False),
        scratch_types=[
            pltpu.VMEM((2, CH), jnp.float32),
            pltpu.VMEM((2, CH), jnp.float32),
            pltpu.VMEM((2, CH), jnp.float32),
            pltpu.VMEM((2, 8, CH), jnp.int32),
            pltpu.VMEM((2, 8, CH), jnp.float32),
            pltpu.VMEM((2, 8, CH, D_OUT), jnp.float32),
            pltpu.VMEM((2, CH, D_OUT), jnp.float32),
            pltpu.SemaphoreType.DMA,
            pltpu.SemaphoreType.DMA,
            pltpu.SemaphoreType.DMA,
            pltpu.SemaphoreType.DMA,
        ],
    )
    def interp(yt, x0, x1, x2, out,
               x0v, x1v, x2v, idxv, wv, rowsv, outv, gs0, gs1, os0, os1):
        wid = lax.axis_index("s") * nc + lax.axis_index("c")
        wbase = wid * qpw
        gsem = (gs0, gs1)
        osem = (os0, os1)

        def stage(ci, b):
            qbase = wbase + ci * CH
            pltpu.sync_copy(x0.at[pl.ds(qbase, CH)], x0v.at[b])
            pltpu.sync_copy(x1.at[pl.ds(qbase, CH)], x1v.at[b])
            pltpu.sync_copy(x2.at[pl.ds(qbase, CH)], x2v.at[b])

            def group_body(g, _):
                sl = pl.ds(g * L, L)
                f0 = jnp.clip(x0v[b, sl], 0.0, float(GRID - 1))
                f1 = jnp.clip(x1v[b, sl], 0.0, float(GRID - 1))
                f2 = jnp.clip(x2v[b, sl], 0.0, float(GRID - 1))
                i0 = jnp.minimum(f0.astype(jnp.int32), GRID - 2)
                i1 = jnp.minimum(f1.astype(jnp.int32), GRID - 2)
                i2 = jnp.minimum(f2.astype(jnp.int32), GRID - 2)
                t0 = f0 - i0.astype(jnp.float32)
                t1 = f1 - i1.astype(jnp.float32)
                t2 = f2 - i2.astype(jnp.float32)
                u0 = 1.0 - t0
                u1 = 1.0 - t1
                u2 = 1.0 - t2
                base = i0 * (GRID * GRID) + i1 * GRID + i2
                a00 = u1 * u2
                a01 = u1 * t2
                a10 = t1 * u2
                a11 = t1 * t2
                ws = (u0 * a00, u0 * a01, u0 * a10, u0 * a11,
                      t0 * a00, t0 * a01, t0 * a10, t0 * a11)
                for c in range(8):
                    idxv[b, c, sl] = base + _CORNER_OFFS[c]
                    wv[b, c, sl] = ws[c]
                return 0

            lax.fori_loop(0, CH // L, group_body, 0)
            for c in range(8):
                pltpu.async_copy(yt.at[idxv.at[b, c]], rowsv.at[b, c], gsem[b])

        def combine(ci, b, k):
            qbase = wbase + ci * CH
            for c in range(8):
                pltpu.make_async_copy(yt.at[idxv.at[b, c]], rowsv.at[b, c],
                                      gsem[b]).wait()

            @pl.when(k > 0)
            def _():
                pltpu.make_async_copy(outv.at[b], out.at[pl.ds(qbase, CH)],
                                      osem[b]).wait()

            def comb_body(g, _):
                sl = pl.ds(g * L, L)
                wvecs = [wv[b, c, sl] for c in range(8)]
                for j in range(L):
                    q = g * L + j
                    w = [wvecs[c][j] for c in range(8)]
                    for h in range(D_OUT // L):
                        hs = pl.ds(h * L, L)
                        acc = w[0] * rowsv[b, 0, q, hs]
                        for c in range(1, 8):
                            acc = acc + w[c] * rowsv[b, c, q, hs]
                        outv[b, q, hs] = acc
                return 0

            lax.fori_loop(0, CH // L, comb_body, 0)
            pltpu.async_copy(outv.at[b], out.at[pl.ds(qbase, CH)], osem[b])

        stage(0, 0)

        def pair_body(k, _):
            stage(2 * k + 1, 1)
            combine(2 * k, 0, k)

            @pl.when(k < n_chunks // 2 - 1)
            def _():
                stage(2 * k + 2, 0)

            combine(2 * k + 1, 1, k)
            return 0

        lax.fori_loop(0, n_chunks // 2, pair_body, 0)
        pltpu.make_async_copy(outv.at[0], out.at[pl.ds(wbase, CH)], os0).wait()
        pltpu.make_async_copy(outv.at[1], out.at[pl.ds(wbase, CH)], os1).wait()

    return interp


def kernel(y, xs0, xs1, xs2, x):
    n_query = x.shape[0]
    yt = y.reshape(GRID * GRID * GRID, D_OUT)
    interp = _make_sc_interp(n_query)
    return interp(yt, x[:, 0], x[:, 1], x[:, 2])


# speedup vs baseline: 2.1717x; 1.2349x over previous
import functools

import jax
import jax.numpy as jnp
from jax import lax
from jax.experimental import pallas as pl
from jax.experimental.pallas import tpu as pltpu
from jax.experimental.pallas import tpu_sc as plsc

D_IN = 3
GRID = 64
D_OUT = 32
CH = 128
L = 16

_CORNER_OFFS = tuple(
    e0 * GRID * GRID + e1 * GRID + e2
    for e0 in (0, 1) for e1 in (0, 1) for e2 in (0, 1)
)


def _make_sc_interp(n_query: int):
    info = plsc.get_sparse_core_info()
    nc, ns = info.num_cores, info.num_subcores
    nw = nc * ns
    assert n_query % (nw * 2 * CH) == 0
    qpw = n_query // nw
    n_chunks = qpw // CH

    mesh = plsc.VectorSubcoreMesh(core_axis_name="c", subcore_axis_name="s")

    @functools.partial(
        pl.kernel,
        out_type=jax.ShapeDtypeStruct((n_query, D_OUT), jnp.float32),
        mesh=mesh,
        compiler_params=pltpu.CompilerParams(use_tc_tiling_on_sc=False),
        scratch_types=[
            pltpu.VMEM((2, CH), jnp.float32),
            pltpu.VMEM((2, CH), jnp.float32),
            pltpu.VMEM((2, CH), jnp.float32),
            pltpu.VMEM((2, 8, CH), jnp.int32),
            pltpu.VMEM((2, 8, CH), jnp.float32),
            pltpu.VMEM((2, 8, CH, D_OUT), jnp.float32),
            pltpu.VMEM((2, CH, D_OUT), jnp.float32),
            pltpu.SemaphoreType.DMA,
            pltpu.SemaphoreType.DMA,
            pltpu.SemaphoreType.DMA,
            pltpu.SemaphoreType.DMA,
            pltpu.SemaphoreType.DMA,
            pltpu.SemaphoreType.DMA,
        ],
    )
    def interp(yt, x0, x1, x2, out,
               x0v, x1v, x2v, idxv, wv, rowsv, outv,
               gs0, gs1, os0, os1, xs0s, xs1s):
        wid = lax.axis_index("s") * nc + lax.axis_index("c")
        wbase = wid * qpw
        gsem = (gs0, gs1)
        osem = (os0, os1)
        xsem = (xs0s, xs1s)

        def load_x(ci, b):
            qbase = wbase + ci * CH
            pltpu.async_copy(x0.at[pl.ds(qbase, CH)], x0v.at[b], xsem[b])
            pltpu.async_copy(x1.at[pl.ds(qbase, CH)], x1v.at[b], xsem[b])
            pltpu.async_copy(x2.at[pl.ds(qbase, CH)], x2v.at[b], xsem[b])

        def wait_x(ci, b):
            qbase = wbase + ci * CH
            pltpu.make_async_copy(x0.at[pl.ds(qbase, CH)], x0v.at[b],
                                  xsem[b]).wait()
            pltpu.make_async_copy(x1.at[pl.ds(qbase, CH)], x1v.at[b],
                                  xsem[b]).wait()
            pltpu.make_async_copy(x2.at[pl.ds(qbase, CH)], x2v.at[b],
                                  xsem[b]).wait()

        def stage(ci, b):
            qbase = wbase + ci * CH

            def group_body(g, _):
                sl = pl.ds(g * L, L)
                f0 = jnp.clip(x0v[b, sl], 0.0, float(GRID - 1))
                f1 = jnp.clip(x1v[b, sl], 0.0, float(GRID - 1))
                f2 = jnp.clip(x2v[b, sl], 0.0, float(GRID - 1))
                i0 = jnp.minimum(f0.astype(jnp.int32), GRID - 2)
                i1 = jnp.minimum(f1.astype(jnp.int32), GRID - 2)
                i2 = jnp.minimum(f2.astype(jnp.int32), GRID - 2)
                t0 = f0 - i0.astype(jnp.float32)
                t1 = f1 - i1.astype(jnp.float32)
                t2 = f2 - i2.astype(jnp.float32)
                u0 = 1.0 - t0
                u1 = 1.0 - t1
                u2 = 1.0 - t2
                base = i0 * (GRID * GRID) + i1 * GRID + i2
                a00 = u1 * u2
                a01 = u1 * t2
                a10 = t1 * u2
                a11 = t1 * t2
                ws = (u0 * a00, u0 * a01, u0 * a10, u0 * a11,
                      t0 * a00, t0 * a01, t0 * a10, t0 * a11)
                for c in range(8):
                    idxv[b, c, sl] = base + _CORNER_OFFS[c]
                    wv[b, c, sl] = ws[c]
                return 0

            lax.fori_loop(0, CH // L, group_body, 0)
            for c in range(8):
                pltpu.async_copy(yt.at[idxv.at[b, c]], rowsv.at[b, c], gsem[b])

        def combine(ci, b, k):
            qbase = wbase + ci * CH
            for c in range(8):
                pltpu.make_async_copy(yt.at[idxv.at[b, c]], rowsv.at[b, c],
                                      gsem[b]).wait()

            @pl.when(k > 0)
            def _():
                pltpu.make_async_copy(outv.at[b], out.at[pl.ds(qbase, CH)],
                                      osem[b]).wait()

            def comb_body(g, _):
                sl = pl.ds(g * L, L)
                wvecs = [wv[b, c, sl] for c in range(8)]
                for j in range(L):
                    q = g * L + j
                    w = [wvecs[c][j] for c in range(8)]
                    for h in range(D_OUT // L):
                        hs = pl.ds(h * L, L)
                        acc = w[0] * rowsv[b, 0, q, hs]
                        for c in range(1, 8):
                            acc = acc + w[c] * rowsv[b, c, q, hs]
                        outv[b, q, hs] = acc
                return 0

            lax.fori_loop(0, CH // L, comb_body, 0)
            pltpu.async_copy(outv.at[b], out.at[pl.ds(qbase, CH)], osem[b])

        load_x(0, 0)
        wait_x(0, 0)
        stage(0, 0)
        load_x(1, 1)

        def pair_body(k, _):
            last = k >= n_chunks // 2 - 1
            wait_x(2 * k + 1, 1)
            stage(2 * k + 1, 1)

            @pl.when(jnp.logical_not(last))
            def _():
                load_x(2 * k + 2, 0)

            combine(2 * k, 0, k)

            @pl.when(jnp.logical_not(last))
            def _():
                wait_x(2 * k + 2, 0)
                stage(2 * k + 2, 0)
                load_x(2 * k + 3, 1)

            combine(2 * k + 1, 1, k)
            return 0

        lax.fori_loop(0, n_chunks // 2, pair_body, 0)
        pltpu.make_async_copy(outv.at[0], out.at[pl.ds(wbase, CH)], os0).wait()
        pltpu.make_async_copy(outv.at[1], out.at[pl.ds(wbase, CH)], os1).wait()

    return interp


def kernel(y, xs0, xs1, xs2, x):
    n_query = x.shape[0]
    yt = y.reshape(GRID * GRID * GRID, D_OUT)
    interp = _make_sc_interp(n_query)
    return interp(yt, x[:, 0], x[:, 1], x[:, 2])
